# baseline (speedup 1.0000x reference)
import jax
import jax.numpy as jnp
from jax.experimental import pallas as pl

_B, _N, _T, _F = 8, 500, 32, 16
_H = 16
_E = 10000
_BN = _B * _N
_ECH = 2000


def _body(xr_ref, ei_ref, Wih0_ref, Whh0_ref, bih0_ref, bhh0_ref,
          Wih1_ref, Whh1_ref, bih1_ref, bhh1_ref,
          gW1_ref, gb1_ref, gW2_ref, gb2_ref,
          L0_ref, lb0_ref, L1_ref, lb1_ref, L2_ref, lb2_ref, L3_ref, lb3_ref,
          out_ref):
    f32 = jnp.float32

    def mm(a, b):
        return jax.lax.dot_general(a, b, (((1,), (0,)), ((), ())),
                                   preferred_element_type=f32)

    m23 = mm(L2_ref[...], L3_ref[...])
    m123 = mm(L1_ref[...], m23)
    m = mm(L0_ref[...], m123)
    g2m = mm(gW2_ref[...], m)
    w_fold = mm(gW1_ref[...], g2m)
    alpha = mm(gb1_ref[...], g2m)
    c_mlp = (mm(lb0_ref[...], m123) + mm(lb1_ref[...], m23)
             + mm(lb2_ref[...], L3_ref[...]) + lb3_ref[...])
    beta = mm(gb2_ref[...], m) + c_mlp

    def mmT(a, b):
        return jax.lax.dot_general(a, b, (((1,), (1,)), ((), ())),
                                   preferred_element_type=f32)

    adj = jnp.zeros((_N, _N), f32)
    iota_e = jax.lax.broadcasted_iota(jnp.int32, (_N, _ECH), 0)
    for k in range(_E // _ECH):
        dst = ei_ref[1:2, k * _ECH:(k + 1) * _ECH]
        src = ei_ref[0:1, k * _ECH:(k + 1) * _ECH]
        oh_dstT = (iota_e == dst).astype(jnp.bfloat16)
        oh_srcT = (iota_e == src).astype(jnp.bfloat16)
        adj = adj + mmT(oh_dstT, oh_srcT)

    eye = (jax.lax.broadcasted_iota(jnp.int32, (_N, _N), 0)
           == jax.lax.broadcasted_iota(jnp.int32, (_N, _N), 1)).astype(f32)
    deg_col = jnp.sum(adj, axis=1, keepdims=True) + 2.0
    deg_row = jnp.sum(eye * deg_col, axis=0, keepdims=True)
    dinv_row = jax.lax.rsqrt(deg_row)
    dinv_col = jax.lax.rsqrt(deg_col)
    ahat = adj * dinv_col * dinv_row + eye * (2.0 / deg_col)
    r_col = jnp.sum(ahat, axis=1, keepdims=True)
    a2 = mm(ahat, ahat)

    xrT = jnp.transpose(xr_ref[...])
    Wc0 = jnp.concatenate([Wih0_ref[...], Whh0_ref[...]], axis=1)
    Wc1 = jnp.concatenate([Wih1_ref[...], Whh1_ref[...]], axis=1)
    b0 = bih0_ref[...] + bhh0_ref[...]
    b1 = bih1_ref[...] + bhh1_ref[...]
    wT = jnp.transpose(w_fold)

    def sig(v):
        return 0.5 * jnp.tanh(0.5 * v) + 0.5

    h1 = jnp.zeros((_H, _BN), f32)
    c1 = jnp.zeros((_H, _BN), f32)
    h2 = jnp.zeros((_H, _BN), f32)
    c2 = jnp.zeros((_H, _BN), f32)
    ys = []
    for t in range(_T):
        xt = xrT[t * _F:(t + 1) * _F]
        S = jnp.concatenate([xt, h1], axis=0)
        Gt = mm(Wc0, S) + b0
        ig = sig(Gt[0:_H])
        fg = sig(Gt[_H:2 * _H])
        gg = jnp.tanh(Gt[2 * _H:3 * _H])
        og = sig(Gt[3 * _H:4 * _H])
        c1 = fg * c1 + ig * gg
        h1 = og * jnp.tanh(c1)

        S2 = jnp.concatenate([h1, h2], axis=0)
        G2 = mm(Wc1, S2) + b1
        i2 = sig(G2[0:_H])
        f2 = sig(G2[_H:2 * _H])
        g2 = jnp.tanh(G2[2 * _H:3 * _H])
        o2 = sig(G2[3 * _H:4 * _H])
        c2 = f2 * c2 + i2 * g2
        h2 = o2 * jnp.tanh(c2)

        ys.append(mm(wT, h2))

    Y = jnp.concatenate(ys, axis=0)
    Yr = jnp.transpose(Y)
    Yb = jnp.reshape(Yr, (_B, _N, _T))

    zs = [mm(a2, Yb[b])[None] for b in range(_B)]
    out = jnp.concatenate(zs, axis=0) + alpha[0, 0] * r_col + beta[0, 0]
    out_ref[...] = out


def kernel(x, edge_index, Wih0, Whh0, bih0, bhh0, Wih1, Whh1, bih1, bhh1,
           gW1, gb1, gW2, gb2, lw0, lb0, lw1, lb1, lw2, lb2, lw3, lb3):
    x2 = x.reshape(_BN, _T * _F)
    ei = edge_index.astype(jnp.int32)

    out = pl.pallas_call(
        _body,
        out_shape=jax.ShapeDtypeStruct((_B, _N, _T), jnp.float32),
    )(x2, ei, Wih0, Whh0,
      bih0.reshape(4 * _H, 1), bhh0.reshape(4 * _H, 1),
      Wih1, Whh1, bih1.reshape(4 * _H, 1), bhh1.reshape(4 * _H, 1),
      gW1, gb1.reshape(1, _H), gW2, gb2.reshape(1, 16),
      lw0.T, lb0.reshape(1, 16), lw1.T, lb1.reshape(1, 8),
      lw2.T, lb2.reshape(1, 4), lw3.T, lb3.reshape(1, 1))

    return out


# speedup vs baseline: 1.1238x; 1.1238x over previous
import jax
import jax.numpy as jnp
from jax.experimental import pallas as pl

_B, _N, _T, _F = 8, 500, 32, 16
_H = 16
_E = 10000
_BN = _B * _N
_ECH = 2000


def _body(xT_ref, ei_ref, Wc0_ref, bih0_ref, bhh0_ref,
          Wc1_ref, bih1_ref, bhh1_ref,
          gW1_ref, gb1_ref, gW2_ref, gb2_ref,
          L0_ref, lb0_ref, L1_ref, lb1_ref, L2_ref, lb2_ref, L3_ref, lb3_ref,
          out_ref):
    f32 = jnp.float32

    def mm(a, b):
        return jax.lax.dot_general(a, b, (((1,), (0,)), ((), ())),
                                   preferred_element_type=f32)

    m23 = mm(L2_ref[...], L3_ref[...])
    m123 = mm(L1_ref[...], m23)
    m = mm(L0_ref[...], m123)
    g2m = mm(gW2_ref[...], m)
    w_fold = mm(gW1_ref[...], g2m)
    alpha = mm(gb1_ref[...], g2m)
    c_mlp = (mm(lb0_ref[...], m123) + mm(lb1_ref[...], m23)
             + mm(lb2_ref[...], L3_ref[...]) + lb3_ref[...])
    beta = mm(gb2_ref[...], m) + c_mlp

    def mmT(a, b):
        return jax.lax.dot_general(a, b, (((1,), (1,)), ((), ())),
                                   preferred_element_type=f32)

    adj = jnp.zeros((_N, _N), f32)
    iota_e = jax.lax.broadcasted_iota(jnp.int32, (_N, _ECH), 0)
    for k in range(_E // _ECH):
        dst = ei_ref[1:2, k * _ECH:(k + 1) * _ECH]
        src = ei_ref[0:1, k * _ECH:(k + 1) * _ECH]
        oh_dstT = (iota_e == dst).astype(jnp.bfloat16)
        oh_srcT = (iota_e == src).astype(jnp.bfloat16)
        adj = adj + mmT(oh_dstT, oh_srcT)

    eye = (jax.lax.broadcasted_iota(jnp.int32, (_N, _N), 0)
           == jax.lax.broadcasted_iota(jnp.int32, (_N, _N), 1)).astype(f32)
    deg_col = jnp.sum(adj, axis=1, keepdims=True) + 2.0
    deg_row = jnp.sum(eye * deg_col, axis=0, keepdims=True)
    dinv_row = jax.lax.rsqrt(deg_row)
    dinv_col = jax.lax.rsqrt(deg_col)
    ahat = adj * dinv_col * dinv_row + eye * (2.0 / deg_col)
    r_col = jnp.sum(ahat, axis=1, keepdims=True)
    a2 = mm(ahat, ahat)

    Wc0 = Wc0_ref[...]
    Wc1 = Wc1_ref[...]
    b0 = bih0_ref[...] + bhh0_ref[...]
    b1 = bih1_ref[...] + bhh1_ref[...]
    wT = jnp.transpose(w_fold)

    def sig(v):
        return 0.5 * jnp.tanh(0.5 * v) + 0.5

    h1 = jnp.zeros((_H, _BN), f32)
    c1 = jnp.zeros((_H, _BN), f32)
    h2 = jnp.zeros((_H, _BN), f32)
    c2 = jnp.zeros((_H, _BN), f32)
    ys = []
    for t in range(_T):
        xt = xT_ref[t]
        S = jnp.concatenate([xt, h1], axis=0)
        Gt = mm(Wc0, S) + b0
        ig = sig(Gt[0:_H])
        fg = sig(Gt[_H:2 * _H])
        gg = jnp.tanh(Gt[2 * _H:3 * _H])
        og = sig(Gt[3 * _H:4 * _H])
        c1 = fg * c1 + ig * gg
        h1 = og * jnp.tanh(c1)

        S2 = jnp.concatenate([h1, h2], axis=0)
        G2 = mm(Wc1, S2) + b1
        i2 = sig(G2[0:_H])
        f2 = sig(G2[_H:2 * _H])
        g2 = jnp.tanh(G2[2 * _H:3 * _H])
        o2 = sig(G2[3 * _H:4 * _H])
        c2 = f2 * c2 + i2 * g2
        h2 = o2 * jnp.tanh(c2)

        ys.append(mm(wT, h2))

    Y = jnp.concatenate(ys, axis=0)
    Yr = jnp.transpose(Y)
    Yb = jnp.reshape(Yr, (_B, _N, _T))

    zs = [mm(a2, Yb[b])[None] for b in range(_B)]
    out = jnp.concatenate(zs, axis=0) + alpha[0, 0] * r_col + beta[0, 0]
    out_ref[...] = out


def kernel(x, edge_index, Wih0, Whh0, bih0, bhh0, Wih1, Whh1, bih1, bhh1,
           gW1, gb1, gW2, gb2, lw0, lb0, lw1, lb1, lw2, lb2, lw3, lb3):
    xT = jnp.transpose(x, (2, 3, 0, 1)).reshape(_T, _F, _BN)
    ei = edge_index.astype(jnp.int32)
    Wc0 = jnp.concatenate([Wih0, Whh0], axis=1)
    Wc1 = jnp.concatenate([Wih1, Whh1], axis=1)

    out = pl.pallas_call(
        _body,
        out_shape=jax.ShapeDtypeStruct((_B, _N, _T), jnp.float32),
    )(xT, ei, Wc0,
      bih0.reshape(4 * _H, 1), bhh0.reshape(4 * _H, 1),
      Wc1, bih1.reshape(4 * _H, 1), bhh1.reshape(4 * _H, 1),
      gW1, gb1.reshape(1, _H), gW2, gb2.reshape(1, 16),
      lw0.T, lb0.reshape(1, 16), lw1.T, lb1.reshape(1, 8),
      lw2.T, lb2.reshape(1, 4), lw3.T, lb3.reshape(1, 1))

    return out


# speedup vs baseline: 1.1935x; 1.0620x over previous
import dataclasses

import jax
import jax.numpy as jnp
from jax import lax
from jax.experimental import pallas as pl
from jax.experimental.pallas import tpu as pltpu
from jax.experimental.pallas import tpu_sc as plsc

_B, _N, _T, _F = 8, 500, 32, 16
_H = 16
_E = 10000
_BN = _B * _N
_NP = 512


def _sc_adj_body(ei_hbm, adj_hbm, ei_ts, accum):
    f32 = jnp.float32
    k = lax.axis_index("core") * 16 + lax.axis_index("subcore")
    base = k * 16

    pltpu.sync_copy(ei_hbm, ei_ts)

    zeros = jnp.zeros((16,), f32)
    for r in range(16):
        @pl.loop(0, _NP, step=16)
        def _(c, r=r):
            accum.at[r, pl.ds(c, 16)][...] = zeros

    ones = jnp.ones((16,), f32)

    @pl.loop(0, _E, step=16)
    def _(i):
        dstv = ei_ts[1, pl.ds(i, 16)]
        srcv = ei_ts[0, pl.ds(i, 16)]
        mask = jnp.logical_and(dstv >= base, dstv < base + 16)
        row = jnp.where(mask, dstv - base, 0)
        plsc.addupdate_scatter(accum, [row, srcv], ones, mask=mask)

    pltpu.sync_copy(accum, adj_hbm.at[pl.ds(base, 16), :])


def _sc_adj(ei):
    cp = pltpu.CompilerParams()
    if "needs_layout_passes" in pltpu.CompilerParams.__dataclass_fields__:
        cp = dataclasses.replace(cp, needs_layout_passes=False)
    return pl.kernel(
        _sc_adj_body,
        out_type=jax.ShapeDtypeStruct((_NP, _NP), jnp.float32),
        mesh=plsc.VectorSubcoreMesh(core_axis_name="core",
                                    subcore_axis_name="subcore"),
        scratch_types=[pltpu.VMEM((2, _E), jnp.int32),
                       pltpu.VMEM((16, _NP), jnp.float32)],
        compiler_params=cp,
    )(ei)


def _lstm_body(xT_ref, Wc0_ref, bih0_ref, bhh0_ref, Wc1_ref, bih1_ref,
               bhh1_ref, gW1_ref, gW2_ref,
               L0_ref, L1_ref, L2_ref, L3_ref, y_ref):
    f32 = jnp.float32

    def mm(a, b):
        return jax.lax.dot_general(a, b, (((1,), (0,)), ((), ())),
                                   preferred_element_type=f32)

    m23 = mm(L2_ref[...], L3_ref[...])
    m123 = mm(L1_ref[...], m23)
    m = mm(L0_ref[...], m123)
    w_fold = mm(gW1_ref[...], mm(gW2_ref[...], m))
    wT = jnp.transpose(w_fold)

    Wc0 = Wc0_ref[...]
    Wc1 = Wc1_ref[...]
    b0 = bih0_ref[...] + bhh0_ref[...]
    b1 = bih1_ref[...] + bhh1_ref[...]

    def sig(v):
        return 0.5 * jnp.tanh(0.5 * v) + 0.5

    h1 = jnp.zeros((_H, _BN), f32)
    c1 = jnp.zeros((_H, _BN), f32)
    h2 = jnp.zeros((_H, _BN), f32)
    c2 = jnp.zeros((_H, _BN), f32)
    ys = []
    for t in range(_T):
        xt = xT_ref[t]
        S = jnp.concatenate([xt, h1], axis=0)
        Gt = mm(Wc0, S) + b0
        ig = sig(Gt[0:_H])
        fg = sig(Gt[_H:2 * _H])
        gg = jnp.tanh(Gt[2 * _H:3 * _H])
        og = sig(Gt[3 * _H:4 * _H])
        c1 = fg * c1 + ig * gg
        h1 = og * jnp.tanh(c1)

        S2 = jnp.concatenate([h1, h2], axis=0)
        G2 = mm(Wc1, S2) + b1
        i2 = sig(G2[0:_H])
        f2 = sig(G2[_H:2 * _H])
        g2 = jnp.tanh(G2[2 * _H:3 * _H])
        o2 = sig(G2[3 * _H:4 * _H])
        c2 = f2 * c2 + i2 * g2
        h2 = o2 * jnp.tanh(c2)

        ys.append(mm(wT, h2))

    y_ref[...] = jnp.concatenate(ys, axis=0)


def _graph_body(y_ref, adjp_ref, gW1_ref, gb1_ref, gW2_ref, gb2_ref,
                L0_ref, lb0_ref, L1_ref, lb1_ref, L2_ref, lb2_ref,
                L3_ref, lb3_ref, out_ref):
    f32 = jnp.float32

    def mm(a, b):
        return jax.lax.dot_general(a, b, (((1,), (0,)), ((), ())),
                                   preferred_element_type=f32)

    m23 = mm(L2_ref[...], L3_ref[...])
    m123 = mm(L1_ref[...], m23)
    m = mm(L0_ref[...], m123)
    g2m = mm(gW2_ref[...], m)
    alpha = mm(gb1_ref[...], g2m)
    c_mlp = (mm(lb0_ref[...], m123) + mm(lb1_ref[...], m23)
             + mm(lb2_ref[...], L3_ref[...]) + lb3_ref[...])
    beta = mm(gb2_ref[...], m) + c_mlp

    adj = adjp_ref[...]
    eye = (jax.lax.broadcasted_iota(jnp.int32, (_NP, _NP), 0)
           == jax.lax.broadcasted_iota(jnp.int32, (_NP, _NP), 1)).astype(f32)
    deg_col = jnp.sum(adj, axis=1, keepdims=True) + 2.0
    deg_row = jnp.sum(eye * deg_col, axis=0, keepdims=True)
    ahat = (adj * jax.lax.rsqrt(deg_col) * jax.lax.rsqrt(deg_row)
            + eye * (2.0 / deg_col))
    r_col = jnp.sum(ahat, axis=1, keepdims=True)
    a2 = mm(ahat, ahat)

    Y = y_ref[...]
    Yr = jnp.transpose(Y)
    Yb = jnp.reshape(Yr, (_B, _N, _T))
    zpad = jnp.zeros((_NP - _N, _T), f32)

    zs = []
    for b in range(_B):
        ybp = jnp.concatenate([Yb[b], zpad], axis=0)
        zs.append(mm(a2, ybp)[0:_N][None])
    out = jnp.concatenate(zs, axis=0) + alpha[0, 0] * r_col[0:_N] + beta[0, 0]
    out_ref[...] = out


def kernel(x, edge_index, Wih0, Whh0, bih0, bhh0, Wih1, Whh1, bih1, bhh1,
           gW1, gb1, gW2, gb2, lw0, lb0, lw1, lb1, lw2, lb2, lw3, lb3):
    xT = jnp.transpose(x, (2, 3, 0, 1)).reshape(_T, _F, _BN)
    ei = edge_index.astype(jnp.int32)
    Wc0 = jnp.concatenate([Wih0, Whh0], axis=1)
    Wc1 = jnp.concatenate([Wih1, Whh1], axis=1)

    adjp = _sc_adj(ei)

    y = pl.pallas_call(
        _lstm_body,
        out_shape=jax.ShapeDtypeStruct((_T, _BN), jnp.float32),
    )(xT, Wc0, bih0.reshape(4 * _H, 1), bhh0.reshape(4 * _H, 1),
      Wc1, bih1.reshape(4 * _H, 1), bhh1.reshape(4 * _H, 1),
      gW1, gW2, lw0.T, lw1.T, lw2.T, lw3.T)

    out = pl.pallas_call(
        _graph_body,
        out_shape=jax.ShapeDtypeStruct((_B, _N, _T), jnp.float32),
    )(y, adjp, gW1, gb1.reshape(1, _H), gW2, gb2.reshape(1, 16),
      lw0.T, lb0.reshape(1, 16), lw1.T, lb1.reshape(1, 8),
      lw2.T, lb2.reshape(1, 4), lw3.T, lb3.reshape(1, 1))

    return out


# speedup vs baseline: 1.2043x; 1.0091x over previous
import dataclasses

import jax
import jax.numpy as jnp
from jax import lax
from jax.experimental import pallas as pl
from jax.experimental.pallas import tpu as pltpu
from jax.experimental.pallas import tpu_sc as plsc

_B, _N, _T, _F = 8, 500, 32, 16
_H = 16
_E = 10000
_BN = _B * _N
_NP = 512


def _sc_adj_body(ei_hbm, adj_hbm, ei_ts, accum):
    f32 = jnp.float32
    k = lax.axis_index("core") * 16 + lax.axis_index("subcore")
    base = k * 16

    pltpu.sync_copy(ei_hbm, ei_ts)

    zeros = jnp.zeros((16,), f32)
    for r in range(16):
        @pl.loop(0, _NP, step=16)
        def _(c, r=r):
            accum.at[r, pl.ds(c, 16)][...] = zeros

    ones = jnp.ones((16,), f32)

    @pl.loop(0, _E, step=16)
    def _(i):
        dstv = ei_ts[1, pl.ds(i, 16)]
        srcv = ei_ts[0, pl.ds(i, 16)]
        mask = jnp.logical_and(dstv >= base, dstv < base + 16)
        row = jnp.where(mask, dstv - base, 0)
        plsc.addupdate_scatter(accum, [row, srcv], ones, mask=mask)

    pltpu.sync_copy(accum, adj_hbm.at[pl.ds(base, 16), :])


def _sc_adj(ei):
    cp = pltpu.CompilerParams()
    if "needs_layout_passes" in pltpu.CompilerParams.__dataclass_fields__:
        cp = dataclasses.replace(cp, needs_layout_passes=False)
    return pl.kernel(
        _sc_adj_body,
        out_type=jax.ShapeDtypeStruct((_NP, _NP), jnp.float32),
        mesh=plsc.VectorSubcoreMesh(core_axis_name="core",
                                    subcore_axis_name="subcore"),
        scratch_types=[pltpu.VMEM((2, _E), jnp.int32),
                       pltpu.VMEM((16, _NP), jnp.float32)],
        compiler_params=cp,
    )(ei)


def _lstm_body(xT_ref, Wp0_ref, Wp1_ref, gW1_ref, gW2_ref,
               L0_ref, L1_ref, L2_ref, L3_ref, y_ref):
    f32 = jnp.float32

    def mm(a, b):
        return jax.lax.dot_general(a, b, (((1,), (0,)), ((), ())),
                                   preferred_element_type=f32)

    m23 = mm(L2_ref[...], L3_ref[...])
    m123 = mm(L1_ref[...], m23)
    m = mm(L0_ref[...], m123)
    w_fold = mm(gW1_ref[...], mm(gW2_ref[...], m))
    wT = jnp.transpose(w_fold)

    Wp0 = Wp0_ref[...]
    Wp1 = Wp1_ref[...]

    def sig(v):
        return 0.5 * jnp.tanh(v) + 0.5

    ones_row = jnp.ones((1, _BN), f32)
    h1 = jnp.zeros((_H, _BN), f32)
    c1 = jnp.zeros((_H, _BN), f32)
    h2 = jnp.zeros((_H, _BN), f32)
    c2 = jnp.zeros((_H, _BN), f32)
    ys = []
    for t in range(_T):
        xt = xT_ref[t]
        S = jnp.concatenate([xt, h1, ones_row], axis=0)
        Gt = mm(Wp0, S)
        ig = sig(Gt[0:_H])
        fg = sig(Gt[_H:2 * _H])
        gg = jnp.tanh(Gt[2 * _H:3 * _H])
        og = sig(Gt[3 * _H:4 * _H])
        c1 = fg * c1 + ig * gg
        h1 = og * jnp.tanh(c1)

        S2 = jnp.concatenate([h1, h2, ones_row], axis=0)
        G2 = mm(Wp1, S2)
        i2 = sig(G2[0:_H])
        f2 = sig(G2[_H:2 * _H])
        g2 = jnp.tanh(G2[2 * _H:3 * _H])
        o2 = sig(G2[3 * _H:4 * _H])
        c2 = f2 * c2 + i2 * g2
        h2 = o2 * jnp.tanh(c2)

        ys.append(mm(wT, h2))

    y_ref[...] = jnp.concatenate(ys, axis=0)


def _graph_body(y_ref, adjp_ref, gW1_ref, gb1_ref, gW2_ref, gb2_ref,
                L0_ref, lb0_ref, L1_ref, lb1_ref, L2_ref, lb2_ref,
                L3_ref, lb3_ref, out_ref):
    f32 = jnp.float32

    def mm(a, b):
        return jax.lax.dot_general(a, b, (((1,), (0,)), ((), ())),
                                   preferred_element_type=f32)

    m23 = mm(L2_ref[...], L3_ref[...])
    m123 = mm(L1_ref[...], m23)
    m = mm(L0_ref[...], m123)
    g2m = mm(gW2_ref[...], m)
    alpha = mm(gb1_ref[...], g2m)
    c_mlp = (mm(lb0_ref[...], m123) + mm(lb1_ref[...], m23)
             + mm(lb2_ref[...], L3_ref[...]) + lb3_ref[...])
    beta = mm(gb2_ref[...], m) + c_mlp

    adj = adjp_ref[...]
    eye = (jax.lax.broadcasted_iota(jnp.int32, (_NP, _NP), 0)
           == jax.lax.broadcasted_iota(jnp.int32, (_NP, _NP), 1)).astype(f32)
    deg_col = jnp.sum(adj, axis=1, keepdims=True) + 2.0
    deg_row = jnp.sum(eye * deg_col, axis=0, keepdims=True)
    ahat = (adj * jax.lax.rsqrt(deg_col) * jax.lax.rsqrt(deg_row)
            + eye * (2.0 / deg_col))
    r_col = jnp.sum(ahat, axis=1, keepdims=True)
    a2 = mm(ahat, ahat)

    Y = y_ref[...]
    Yr = jnp.transpose(Y)
    Yb = jnp.reshape(Yr, (_B, _N, _T))
    zpad = jnp.zeros((_NP - _N, _T), f32)

    zs = []
    for b in range(_B):
        ybp = jnp.concatenate([Yb[b], zpad], axis=0)
        zs.append(mm(a2, ybp)[0:_N][None])
    out = jnp.concatenate(zs, axis=0) + alpha[0, 0] * r_col[0:_N] + beta[0, 0]
    out_ref[...] = out


def kernel(x, edge_index, Wih0, Whh0, bih0, bhh0, Wih1, Whh1, bih1, bhh1,
           gW1, gb1, gW2, gb2, lw0, lb0, lw1, lb1, lw2, lb2, lw3, lb3):
    xT = jnp.transpose(x, (2, 3, 0, 1)).reshape(_T, _F, _BN)
    ei = edge_index.astype(jnp.int32)
    rs = jnp.concatenate([jnp.full((2 * _H, 1), 0.5, jnp.float32),
                          jnp.ones((_H, 1), jnp.float32),
                          jnp.full((_H, 1), 0.5, jnp.float32)], axis=0)
    Wp0 = jnp.concatenate([Wih0, Whh0, (bih0 + bhh0).reshape(4 * _H, 1)],
                          axis=1) * rs
    Wp1 = jnp.concatenate([Wih1, Whh1, (bih1 + bhh1).reshape(4 * _H, 1)],
                          axis=1) * rs

    adjp = _sc_adj(ei)

    y = pl.pallas_call(
        _lstm_body,
        out_shape=jax.ShapeDtypeStruct((_T, _BN), jnp.float32),
    )(xT, Wp0, Wp1, gW1, gW2, lw0.T, lw1.T, lw2.T, lw3.T)

    out = pl.pallas_call(
        _graph_body,
        out_shape=jax.ShapeDtypeStruct((_B, _N, _T), jnp.float32),
    )(y, adjp, gW1, gb1.reshape(1, _H), gW2, gb2.reshape(1, 16),
      lw0.T, lb0.reshape(1, 16), lw1.T, lb1.reshape(1, 8),
      lw2.T, lb2.reshape(1, 4), lw3.T, lb3.reshape(1, 1))

    return out
